# Initial kernel scaffold; baseline (speedup 1.0000x reference)
import jax
import jax.numpy as jnp
from jax.experimental import pallas as pl


def kernel(x, edge_attr, W_ei, b_ei, W_conv, b_conv, W_en, b_en, W_f1, b_f1, W_f2, b_f2, edge_index, batch, atom_origin_type):
    raise NotImplementedError("write your pallas kernel here")


# speedup vs baseline: 1.0288x; 1.0288x over previous
import functools
import jax
import jax.numpy as jnp
from jax import lax
from jax.experimental import pallas as pl
from jax.experimental.pallas import tpu as pltpu

N = 10000
E = 320000
DF = 128
DE = 16
H = 256
DEPTH = 4
G = 256

BE = 2560
_f32 = jnp.float32
PREC = lax.Precision.DEFAULT


def _dot(a, b, prec=None):
    return jnp.dot(a, b, preferred_element_type=_f32,
                   precision=PREC if prec is None else prec)


def _init_body(xr_ref, ea_ref, wx_ref, we_ref, b_ref, h0_ref):
    h0_ref[...] = jax.nn.relu(
        _dot(xr_ref[...], wx_ref[...])
        + _dot(ea_ref[...], we_ref[...])
        + b_ref[...]
    )


def _pair_swap(g):
    up = jnp.concatenate([g[1:], g[:1]], axis=0)
    dn = jnp.concatenate([g[-1:], g[:-1]], axis=0)
    par = lax.broadcasted_iota(jnp.int32, (g.shape[0], 1), 0) % 2
    return jnp.where(par == 0, up, dn)


def _layer_body(gath_ref, h_ref, h0_ref, b_ref, w_ref, o_ref):
    t = gath_ref[...] - _pair_swap(h_ref[...])
    o_ref[...] = jax.nn.relu(_dot(t, w_ref[...]) + b_ref[...] + h0_ref[...])


def _final_body(x_ref, s_ref, wx_ref, ws_ref, ben_ref, batch_ref, wf1_ref,
                bf1_ref, wf2_ref, bf2_ref, o_ref):
    hn = jax.nn.relu(
        _dot(x_ref[...], wx_ref[...])
        + _dot(s_ref[...], ws_ref[...])
        + ben_ref[...]
    )
    gid = lax.broadcasted_iota(jnp.int32, (G, N), 0)
    oh = (batch_ref[...] == gid).astype(_f32)
    pooled = _dot(oh, hn, prec=lax.Precision.HIGHEST)
    f1 = jax.nn.relu(_dot(pooled, wf1_ref[...]) + bf1_ref[...])
    o_ref[...] = _dot(f1, wf2_ref[...]) + bf2_ref[...]


def _edge_spec():
    return pl.BlockSpec((BE, H), lambda i: (i, 0))


def _whole(shape):
    return pl.BlockSpec(shape, lambda i: tuple(0 for _ in shape))


def _tc_init(xr, edge_attr, wx, we, b_ei):
    return pl.pallas_call(
        _init_body,
        grid=(E // BE,),
        in_specs=[
            pl.BlockSpec((BE, DF), lambda i: (i, 0)),
            pl.BlockSpec((BE, DE), lambda i: (i, 0)),
            _whole((DF, H)),
            _whole((DE, H)),
            _whole((1, H)),
        ],
        out_specs=_edge_spec(),
        out_shape=jax.ShapeDtypeStruct((E, H), _f32),
    )(xr, edge_attr, wx, we, b_ei)


def _tc_layer(gath, h, h0, b, w):
    return pl.pallas_call(
        _layer_body,
        grid=(E // BE,),
        in_specs=[
            _edge_spec(),
            _edge_spec(),
            _edge_spec(),
            _whole((1, H)),
            _whole((H, H)),
        ],
        out_specs=_edge_spec(),
        out_shape=jax.ShapeDtypeStruct((E, H), _f32),
    )(gath, h, h0, b, w)


def _tc_final(x, s, wx, ws, b_en, batch2d, wf1, bf1, wf2, bf2):
    return pl.pallas_call(
        _final_body,
        out_shape=jax.ShapeDtypeStruct((G, 1), _f32),
    )(x, s, wx, ws, b_en, batch2d, wf1, bf1, wf2, bf2)


def kernel(x, edge_attr, W_ei, b_ei, W_conv, b_conv, W_en, b_en, W_f1, b_f1,
           W_f2, b_f2, edge_index, batch, atom_origin_type):
    row = edge_index[0].astype(jnp.int32)
    col = edge_index[1].astype(jnp.int32)
    b_ei2 = b_ei.reshape(1, H)
    b_en2 = b_en.reshape(1, H)
    bf1 = b_f1.reshape(1, H)
    bf2 = b_f2.reshape(1, 1)
    batch2d = batch.astype(jnp.int32).reshape(1, N)

    xr = x[row]
    h = _tc_init(xr, edge_attr, W_ei[:DF], W_ei[DF:], b_ei2)
    h0 = h
    for l in range(DEPTH):
        a = jax.ops.segment_sum(h, col, num_segments=N)
        gath = a[row]
        h = _tc_layer(gath, h, h0, b_conv[l].reshape(1, H), W_conv[l])
    s = jax.ops.segment_sum(h, col, num_segments=N)
    out = _tc_final(x, s, W_en[:DF], W_en[DF:], b_en2, batch2d, W_f1, bf1,
                    W_f2, bf2)
    return out.reshape(G)


# speedup vs baseline: 2.2951x; 2.2309x over previous
import functools
import jax
import jax.numpy as jnp
from jax import lax
from jax.experimental import pallas as pl
from jax.experimental.pallas import tpu as pltpu
from jax.experimental.pallas import tpu_sc as plsc

N = 10000
E = 320000
DF = 128
DE = 16
H = 256
HH = 128
DEPTH = 4
G = 256

BE = 2560
_f32 = jnp.float32
_i32 = jnp.int32
PREC = lax.Precision.DEFAULT


def _dot(a, b, prec=None):
    return jnp.dot(a, b, preferred_element_type=_f32,
                   precision=PREC if prec is None else prec)


def _cat(ref):
    return jnp.concatenate([ref[0], ref[1]], axis=1)


def _init_body(xr_ref, ea_ref, wx_ref, we_ref, b_ref, h0_ref):
    h0 = jax.nn.relu(
        _dot(xr_ref[...], wx_ref[...])
        + _dot(ea_ref[...], we_ref[...])
        + b_ref[...]
    )
    h0_ref[0] = h0[:, :HH]
    h0_ref[1] = h0[:, HH:]


def _pair_swap(g):
    up = jnp.concatenate([g[1:], g[:1]], axis=0)
    dn = jnp.concatenate([g[-1:], g[:-1]], axis=0)
    par = lax.broadcasted_iota(jnp.int32, (g.shape[0], 1), 0) % 2
    return jnp.where(par == 0, up, dn)


def _layer_body(gath_ref, h_ref, h0_ref, b_ref, w_ref, o_ref):
    t = _cat(gath_ref) - _pair_swap(_cat(h_ref))
    o = jax.nn.relu(_dot(t, w_ref[...]) + b_ref[...] + _cat(h0_ref))
    o_ref[0] = o[:, :HH]
    o_ref[1] = o[:, HH:]


def _final_body(x_ref, s_ref, wx_ref, ws_ref, ben_ref, batch_ref, wf1_ref,
                bf1_ref, wf2_ref, bf2_ref, o_ref):
    hn = jax.nn.relu(
        _dot(x_ref[...], wx_ref[...])
        + _dot(_cat(s_ref), ws_ref[...])
        + ben_ref[...]
    )
    gid = lax.broadcasted_iota(jnp.int32, (G, N), 0)
    oh = (batch_ref[...] == gid).astype(_f32)
    pooled = _dot(oh, hn, prec=lax.Precision.HIGHEST)
    f1 = jax.nn.relu(_dot(pooled, wf1_ref[...]) + bf1_ref[...])
    o_ref[...] = _dot(f1, wf2_ref[...]) + bf2_ref[...]


def _split_spec():
    return pl.BlockSpec((2, BE, HH), lambda i: (0, i, 0))


def _whole(shape):
    return pl.BlockSpec(shape, lambda *i: tuple(0 for _ in shape))


def _tc_init(xr, edge_attr, wx, we, b_ei):
    return pl.pallas_call(
        _init_body,
        grid=(E // BE,),
        in_specs=[
            pl.BlockSpec((BE, DF), lambda i: (i, 0)),
            pl.BlockSpec((BE, DE), lambda i: (i, 0)),
            _whole((DF, H)),
            _whole((DE, H)),
            _whole((1, H)),
        ],
        out_specs=_split_spec(),
        out_shape=jax.ShapeDtypeStruct((2, E, HH), _f32),
    )(xr, edge_attr, wx, we, b_ei)


def _tc_layer(gath, h, h0, b, w):
    return pl.pallas_call(
        _layer_body,
        grid=(E // BE,),
        in_specs=[
            _split_spec(),
            _split_spec(),
            _split_spec(),
            _whole((1, H)),
            _whole((H, H)),
        ],
        out_specs=_split_spec(),
        out_shape=jax.ShapeDtypeStruct((2, E, HH), _f32),
    )(gath, h, h0, b, w)


def _tc_final(x, s, wx, ws, b_en, batch2d, wf1, bf1, wf2, bf2):
    return pl.pallas_call(
        _final_body,
        in_specs=[
            _whole((N, DF)),
            pl.BlockSpec((2, N, HH), lambda *_: (0, 0, 0)),
            _whole((DF, H)),
            _whole((H, H)),
            _whole((1, H)),
            _whole((1, N)),
            _whole((H, H)),
            _whole((1, H)),
            _whole((H, 1)),
            _whole((1, 1)),
        ],
        out_specs=_whole((G, 1)),
        out_shape=jax.ShapeDtypeStruct((G, 1), _f32),
    )(x, s, wx, ws, b_en, batch2d, wf1, bf1, wf2, bf2)


MESH = plsc.VectorSubcoreMesh(core_axis_name="c", subcore_axis_name="s")
CHK = 128
EPT = E // 16
NFULL = EPT // CHK
TAIL = EPT - NFULL * CHK
ZR = 624
ZTAIL = N - 16 * ZR
EPW = E // 32
NF32 = EPW // CHK
TAIL32 = EPW - NF32 * CHK


def _sc_scatter_gather_body(h2, col, rowi, zeros, gath, A, idxb, gbuf, idxt,
                            gbuft, sem):
    c = lax.axis_index("c")
    s = lax.axis_index("s")
    pltpu.sync_copy(zeros.at[pl.ds(0, ZR)], A.at[pl.ds(s * ZR, ZR)])

    @pl.when(s == 15)
    def _():
        pltpu.sync_copy(zeros.at[pl.ds(0, ZTAIL)], A.at[pl.ds(16 * ZR, ZTAIL)])

    plsc.subcore_barrier()
    base = s * EPT

    def scat(i, carry):
        e0 = base + i * CHK
        pltpu.sync_copy(col.at[pl.ds(e0, CHK)], idxb)
        pltpu.sync_copy(h2.at[c, pl.ds(e0, CHK)], gbuf)
        pltpu.sync_copy(gbuf, A.at[idxb], add=True)
        return carry

    lax.fori_loop(0, NFULL, scat, 0)
    e0 = base + NFULL * CHK
    pltpu.sync_copy(col.at[pl.ds(e0, TAIL)], idxt)
    pltpu.sync_copy(h2.at[c, pl.ds(e0, TAIL)], gbuft)
    pltpu.sync_copy(gbuft, A.at[idxt], add=True)
    plsc.subcore_barrier()

    def gat(i, carry):
        e0 = base + i * CHK
        pltpu.sync_copy(rowi.at[pl.ds(e0, CHK)], idxb)
        pltpu.sync_copy(A.at[idxb], gbuf)
        pltpu.sync_copy(gbuf, gath.at[c, pl.ds(e0, CHK)])
        return carry

    lax.fori_loop(0, NFULL, gat, 0)
    e0 = base + NFULL * CHK
    pltpu.sync_copy(rowi.at[pl.ds(e0, TAIL)], idxt)
    pltpu.sync_copy(A.at[idxt], gbuft)
    pltpu.sync_copy(gbuft, gath.at[c, pl.ds(e0, TAIL)])


def _sc_scatter_out_body(h2, col, zeros, s_out, A, idxb, gbuf, idxt, gbuft,
                         sem):
    c = lax.axis_index("c")
    s = lax.axis_index("s")
    pltpu.sync_copy(zeros.at[pl.ds(0, ZR)], A.at[pl.ds(s * ZR, ZR)])

    @pl.when(s == 15)
    def _():
        pltpu.sync_copy(zeros.at[pl.ds(0, ZTAIL)], A.at[pl.ds(16 * ZR, ZTAIL)])

    plsc.subcore_barrier()
    base = s * EPT

    def scat(i, carry):
        e0 = base + i * CHK
        pltpu.sync_copy(col.at[pl.ds(e0, CHK)], idxb)
        pltpu.sync_copy(h2.at[c, pl.ds(e0, CHK)], gbuf)
        pltpu.sync_copy(gbuf, A.at[idxb], add=True)
        return carry

    lax.fori_loop(0, NFULL, scat, 0)
    e0 = base + NFULL * CHK
    pltpu.sync_copy(col.at[pl.ds(e0, TAIL)], idxt)
    pltpu.sync_copy(h2.at[c, pl.ds(e0, TAIL)], gbuft)
    pltpu.sync_copy(gbuft, A.at[idxt], add=True)
    plsc.subcore_barrier()
    pltpu.sync_copy(A.at[pl.ds(s * ZR, ZR)], s_out.at[c, pl.ds(s * ZR, ZR)])

    @pl.when(s == 15)
    def _():
        pltpu.sync_copy(A.at[pl.ds(16 * ZR, ZTAIL)],
                        s_out.at[c, pl.ds(16 * ZR, ZTAIL)])


def _sc_gatherx_body(x_hbm, rowi, xr, idxb, gbuf, idxt, gbuft, sem):
    c = lax.axis_index("c")
    s = lax.axis_index("s")
    wid = s * 2 + c
    base = wid * EPW

    def gat(i, carry):
        e0 = base + i * CHK
        pltpu.sync_copy(rowi.at[pl.ds(e0, CHK)], idxb)
        pltpu.async_copy(x_hbm.at[idxb], gbuf, sem).wait()
        pltpu.sync_copy(gbuf, xr.at[pl.ds(e0, CHK)])
        return carry

    lax.fori_loop(0, NF32, gat, 0)
    e0 = base + NF32 * CHK
    pltpu.sync_copy(rowi.at[pl.ds(e0, TAIL32)], idxt)
    pltpu.async_copy(x_hbm.at[idxt], gbuft, sem).wait()
    pltpu.sync_copy(gbuft, xr.at[pl.ds(e0, TAIL32)])


def _sc_scatter_gather(h2, col, rowi, zeros):
    return pl.kernel(
        _sc_scatter_gather_body,
        mesh=MESH,
        out_type=jax.ShapeDtypeStruct((2, E, HH), _f32),
        scratch_types=[
            pltpu.VMEM_SHARED((N, HH), _f32),
            pltpu.VMEM((CHK,), _i32),
            pltpu.VMEM((CHK, HH), _f32),
            pltpu.VMEM((TAIL,), _i32),
            pltpu.VMEM((TAIL, HH), _f32),
            pltpu.SemaphoreType.DMA,
        ],
    )(h2, col, rowi, zeros)


def _sc_scatter_out(h2, col, zeros):
    return pl.kernel(
        _sc_scatter_out_body,
        mesh=MESH,
        out_type=jax.ShapeDtypeStruct((2, N, HH), _f32),
        scratch_types=[
            pltpu.VMEM_SHARED((N, HH), _f32),
            pltpu.VMEM((CHK,), _i32),
            pltpu.VMEM((CHK, HH), _f32),
            pltpu.VMEM((TAIL,), _i32),
            pltpu.VMEM((TAIL, HH), _f32),
            pltpu.SemaphoreType.DMA,
        ],
    )(h2, col, zeros)


def _sc_gatherx(x, rowi):
    return pl.kernel(
        _sc_gatherx_body,
        mesh=MESH,
        out_type=jax.ShapeDtypeStruct((E, DF), _f32),
        scratch_types=[
            pltpu.VMEM((CHK,), _i32),
            pltpu.VMEM((CHK, DF), _f32),
            pltpu.VMEM((TAIL32,), _i32),
            pltpu.VMEM((TAIL32, DF), _f32),
            pltpu.SemaphoreType.DMA,
        ],
    )(x, rowi)


def kernel(x, edge_attr, W_ei, b_ei, W_conv, b_conv, W_en, b_en, W_f1, b_f1,
           W_f2, b_f2, edge_index, batch, atom_origin_type):
    row = edge_index[0].astype(jnp.int32)
    col = edge_index[1].astype(jnp.int32)
    b_ei2 = b_ei.reshape(1, H)
    b_en2 = b_en.reshape(1, H)
    bf1 = b_f1.reshape(1, H)
    bf2 = b_f2.reshape(1, 1)
    batch2d = batch.astype(jnp.int32).reshape(1, N)
    zeros = jnp.zeros((ZR, HH), _f32)

    xr = _sc_gatherx(x, row)
    h = _tc_init(xr, edge_attr, W_ei[:DF], W_ei[DF:], b_ei2)
    h0 = h
    for l in range(DEPTH):
        gath = _sc_scatter_gather(h, col, row, zeros)
        h = _tc_layer(gath, h, h0, b_conv[l].reshape(1, H), W_conv[l])
    s = _sc_scatter_out(h, col, zeros)
    out = _tc_final(x, s, W_en[:DF], W_en[DF:], b_en2, batch2d, W_f1, bf1,
                    W_f2, bf2)
    return out.reshape(G)
